# baseline (speedup 1.0000x reference)
import functools

import jax
import jax.numpy as jnp
from jax import lax
from jax.experimental import pallas as pl
from jax.experimental.pallas import tpu as pltpu
from jax.experimental.pallas import tpu_sc as plsc

_T_CHUNK = 32
_T_NBUF = 3
_T_LA = 2
_S_CHUNK = 128
_S_NBUF = 4
_S_LA = 2


def _ring(load, store, n, nbuf, la):
    waited = [False] * n
    for j in range(min(la, n)):
        load(j).start()
    for i in range(n):
        load(i).wait()
        store(i).start()
        if i + la < n:
            f = i + la - nbuf
            if f >= 0:
                store(f).wait()
                waited[f] = True
            load(i + la).start()
    for i in range(n):
        if not waited[i]:
            store(i).wait()


def kernel(x, pos_table):
    seq_len = x.shape[1]
    emb_dim = pos_table.shape[1]

    info = plsc.get_sparse_core_info()
    num_cores, num_subcores = info.num_cores, info.num_subcores
    rows_per_sc = seq_len // num_cores
    n_stream_tiles = num_subcores - 1
    rows_per_tile = ((rows_per_sc // 2) // n_stream_tiles // _T_CHUNK) * _T_CHUNK
    stream_rows = rows_per_tile * n_stream_tiles
    spmem_rows = rows_per_sc - stream_rows
    assert seq_len % num_cores == 0 and rows_per_tile > 0
    assert spmem_rows % _S_CHUNK == 0
    t_chunks = rows_per_tile // _T_CHUNK
    s_chunks = spmem_rows // _S_CHUNK

    mesh = plsc.VectorSubcoreMesh(core_axis_name="c", subcore_axis_name="s")

    @functools.partial(
        pl.kernel,
        mesh=mesh,
        out_type=jax.ShapeDtypeStruct((seq_len, emb_dim), jnp.float32),
        scratch_types=(
            [pltpu.VMEM((_T_CHUNK, emb_dim), jnp.float32) for _ in range(_T_NBUF)]
            + [pltpu.VMEM_SHARED((_S_CHUNK, emb_dim), jnp.float32)
               for _ in range(_S_NBUF)]
            + [pltpu.SemaphoreType.DMA, pltpu.SemaphoreType.DMA]
        ),
    )
    def copy_rows(table_hbm, out_hbm, *rest):
        tbufs = rest[:_T_NBUF]
        sbufs = rest[_T_NBUF:_T_NBUF + _S_NBUF]
        lsem, ssem = rest[_T_NBUF + _S_NBUF:]
        c = lax.axis_index("c")
        s = lax.axis_index("s")
        sc_base = c * rows_per_sc

        @pl.when(s > 0)
        def _stream_path():
            base = sc_base + (s - 1) * rows_per_tile

            def load(i):
                return pltpu.make_async_copy(
                    table_hbm.at[pl.ds(base + i * _T_CHUNK, _T_CHUNK)],
                    tbufs[i % _T_NBUF], lsem)

            def store(i):
                return pltpu.make_async_copy(
                    tbufs[i % _T_NBUF],
                    out_hbm.at[pl.ds(base + i * _T_CHUNK, _T_CHUNK)], ssem)

            _ring(load, store, t_chunks, _T_NBUF, _T_LA)

        @pl.when(s == 0)
        def _spmem_path():
            base = sc_base + stream_rows

            def load(i):
                return pltpu.make_async_copy(
                    table_hbm.at[pl.ds(base + i * _S_CHUNK, _S_CHUNK)],
                    sbufs[i % _S_NBUF], lsem)

            def store(i):
                return pltpu.make_async_copy(
                    sbufs[i % _S_NBUF],
                    out_hbm.at[pl.ds(base + i * _S_CHUNK, _S_CHUNK)], ssem)

            _ring(load, store, s_chunks, _S_NBUF, _S_LA)

    return copy_rows(pos_table)[None]


# speedup vs baseline: 1.0313x; 1.0313x over previous
import functools

import jax
import jax.numpy as jnp
from jax import lax
from jax.experimental import pallas as pl
from jax.experimental.pallas import tpu as pltpu
from jax.experimental.pallas import tpu_sc as plsc

_CHUNK_ROWS = 32
_NBUF = 3
_LA = 2


def kernel(x, pos_table):
    seq_len = x.shape[1]
    emb_dim = pos_table.shape[1]

    info = plsc.get_sparse_core_info()
    num_cores, num_subcores = info.num_cores, info.num_subcores
    num_workers = num_cores * num_subcores
    assert seq_len % (num_workers * _CHUNK_ROWS) == 0
    rows_per_worker = seq_len // num_workers
    nchunks = rows_per_worker // _CHUNK_ROWS

    mesh = plsc.VectorSubcoreMesh(core_axis_name="c", subcore_axis_name="s")

    @functools.partial(
        pl.kernel,
        mesh=mesh,
        out_type=jax.ShapeDtypeStruct((seq_len, emb_dim), jnp.float32),
        scratch_types=(
            [pltpu.VMEM((_CHUNK_ROWS, emb_dim), jnp.float32) for _ in range(_NBUF)]
            + [pltpu.SemaphoreType.DMA, pltpu.SemaphoreType.DMA]
        ),
    )
    def copy_rows(table_hbm, out_hbm, *rest):
        bufs, (lsem, ssem) = rest[:_NBUF], rest[_NBUF:]
        wid = lax.axis_index("s") * num_cores + lax.axis_index("c")
        base = wid * rows_per_worker

        def load(i):
            return pltpu.make_async_copy(
                table_hbm.at[pl.ds(base + i * _CHUNK_ROWS, _CHUNK_ROWS)],
                bufs[i % _NBUF], lsem)

        def store(i):
            return pltpu.make_async_copy(
                bufs[i % _NBUF],
                out_hbm.at[pl.ds(base + i * _CHUNK_ROWS, _CHUNK_ROWS)], ssem)

        store_waited = [False] * nchunks
        for j in range(min(_LA, nchunks)):
            load(j).start()
        for i in range(nchunks):
            load(i).wait()
            store(i).start()
            if i + _LA < nchunks:
                f = i + _LA - _NBUF
                if f >= 0:
                    store(f).wait()
                    store_waited[f] = True
                load(i + _LA).start()
        for i in range(nchunks):
            if not store_waited[i]:
                store(i).wait()

    return copy_rows(pos_table)[None]
